# Initial kernel scaffold; baseline (speedup 1.0000x reference)
import jax
import jax.numpy as jnp
from jax.experimental import pallas as pl


def kernel(S, J, C, D, P):
    raise NotImplementedError("write your pallas kernel here")


# speedup vs baseline: 7.5614x; 7.5614x over previous
import functools

import jax
import jax.numpy as jnp
from jax import lax
from jax.experimental import pallas as pl
from jax.experimental.pallas import tpu as pltpu

_IOU = 0.5
_BIG = 1e9


def _merge_kernel(j_ref, rr_ref, rc_ref, cd_ref, mcd_ref, stats_ref, covu_ref):
    N = j_ref.shape[0]
    _BLK = min(256, N)
    rr = rr_ref[...]
    rc = rc_ref[...]

    def build_blk(b, _):
        i0 = pl.multiple_of(b * _BLK, _BLK)
        jb = j_ref[pl.ds(i0, _BLK), :]
        rcb = rc_ref[pl.ds(i0, _BLK), :]
        covu_ref[pl.ds(i0, _BLK), :] = jnp.where(
            (jb >= _IOU) & (rcb < rr), 1.0, 0.0)
        return 0
    lax.fori_loop(0, N // _BLK, build_blk, 0)

    def cond(c):
        return c[1]

    def body(c):
        h, _ = c
        a = jnp.dot(h, covu_ref[...], preferred_element_type=jnp.float32)
        hn = jnp.where(a > 0.0, 0.0, 1.0)
        return hn, jnp.any(hn != h)

    h, _ = lax.while_loop(
        cond, body, (jnp.ones((1, N), jnp.float32), jnp.bool_(True)))

    g = jnp.where(h > 0.0, rr, _BIG)

    def first_blk(b, _):
        i0 = pl.multiple_of(b * _BLK, _BLK)
        jb = j_ref[pl.ds(i0, _BLK), :]
        tb = jnp.where(jb >= _IOU, g, _BIG)
        fb = jnp.min(tb, axis=1, keepdims=True)
        covu_ref[pl.ds(i0, _BLK), :] = jnp.where(fb == rr, 1.0, 0.0)
        return 0
    lax.fori_loop(0, N // _BLK, first_blk, 0)

    mt = covu_ref[...]
    cnt = jnp.sum(mt, axis=0, keepdims=True)
    ssum = lax.dot_general(
        mt, cd_ref[...], (((0,), (0,)), ((), ())),
        preferred_element_type=jnp.float32)
    inv = jnp.where(cnt > 0.0, 1.0 / jnp.maximum(cnt, 1.0), 0.0)

    def scale_blk(b, _):
        i0 = pl.multiple_of(b * _BLK, _BLK)
        covu_ref[pl.ds(i0, _BLK), :] = covu_ref[pl.ds(i0, _BLK), :] * inv
        return 0
    lax.fori_loop(0, N // _BLK, scale_blk, 0)
    mcd_ref[...] = jnp.dot(covu_ref[...], ssum,
                           preferred_element_type=jnp.float32)

    num_heads = jnp.sum(jnp.where(cnt > 0.0, 1.0, 0.0))
    max_clique = jnp.max(cnt)
    min_clique = jnp.min(jnp.where(cnt > 0.0, cnt, _BIG))
    lane = lax.broadcasted_iota(jnp.int32, (1, 128), 1)
    stats_ref[...] = (num_heads * (lane == 0).astype(jnp.float32)
                      + max_clique * (lane == 1).astype(jnp.float32)
                      + min_clique * (lane == 2).astype(jnp.float32))


def _merge_pallas(J, rr, rc, CD):
    N, K2 = CD.shape
    vmem_limit = int(min(
        2 * N * N * 4 + N * N * 4 + 6 * N * K2 * 4 + (4 << 20), 60 << 20))
    out_shape = (
        jax.ShapeDtypeStruct((N, K2), jnp.float32),
        jax.ShapeDtypeStruct((1, 128), jnp.float32),
    )
    return pl.pallas_call(
        _merge_kernel,
        out_shape=out_shape,
        grid=(1,),
        in_specs=[
            pl.BlockSpec((N, N), lambda i: (0, 0)),
            pl.BlockSpec((1, N), lambda i: (0, 0)),
            pl.BlockSpec((N, 1), lambda i: (0, 0)),
            pl.BlockSpec((N, K2), lambda i: (0, 0)),
        ],
        out_specs=(
            pl.BlockSpec((N, K2), lambda i: (0, 0)),
            pl.BlockSpec((1, 128), lambda i: (0, 0)),
        ),
        scratch_shapes=[pltpu.VMEM((N, N), jnp.float32)],
        compiler_params=pltpu.CompilerParams(
            dimension_semantics=("arbitrary",),
            vmem_limit_bytes=vmem_limit),
    )(J, rr, rc, CD)


def kernel(S, J, C, D, P):
    N = S.shape[0]
    K = C.shape[1]

    order = jnp.argsort(-S)
    rank = jnp.zeros((N,), jnp.int32).at[order].set(
        jnp.arange(N, dtype=jnp.int32))
    rank_f = rank.astype(jnp.float32)
    rr = rank_f.reshape(1, N)
    rc = rank_f.reshape(N, 1)
    CD = jnp.concatenate(
        [C.astype(jnp.float32), D.astype(jnp.float32)], axis=1)

    MCD, stats = _merge_pallas(J.astype(jnp.float32), rr, rc, CD)

    MC = MCD[:, :K].astype(C.dtype)
    MD = MCD[:, K:].astype(D.dtype)

    num_heads = stats[0, 0].astype(jnp.int32)
    max_clique = stats[0, 1].astype(jnp.int32)
    min_clique = stats[0, 2].astype(jnp.int32)
    P_new = (P.at[2].add(1)
              .at[5].add(num_heads)
              .at[6].add(max_clique)
              .at[7].add(min_clique))
    return MC, MD, P_new


# speedup vs baseline: 8.2075x; 1.0854x over previous
import jax
import jax.numpy as jnp
from jax import lax
from jax.experimental import pallas as pl
from jax.experimental.pallas import tpu as pltpu

_IOU = 0.5
_BIG = 1e9


def _merge_kernel(j_ref, sr_ref, sc_ref, cd_ref, mcd_ref, stats_ref, sup_ref):
    N = j_ref.shape[0]
    BLK = min(256, N)
    sr = sr_ref[...]
    ir = lax.broadcasted_iota(jnp.int32, (1, N), 1)
    ibig = jnp.int32(1 << 30)

    def build_blk(b, _):
        i0 = pl.multiple_of(b * BLK, BLK)
        jb = j_ref[pl.ds(i0, BLK), :]
        scb = sc_ref[pl.ds(i0, BLK), :]
        icb = lax.broadcasted_iota(jnp.int32, (BLK, 1), 0) + i0
        earlier = (scb > sr) | ((scb == sr) & (icb < ir))
        sup_ref[pl.ds(i0, BLK), :] = jnp.where(
            (jb >= _IOU) & earlier, 1.0, 0.0).astype(jnp.bfloat16)
        return 0
    lax.fori_loop(0, N // BLK, build_blk, 0)

    def cond(c):
        return c[1]

    def body(c):
        h, _ = c
        a = jnp.dot(h.astype(jnp.bfloat16), sup_ref[...],
                    preferred_element_type=jnp.float32)
        hn = jnp.where(a > 0.0, 0.0, 1.0)
        return hn, jnp.any(hn != h)

    h, _ = lax.while_loop(
        cond, body, (jnp.ones((1, N), jnp.float32), jnp.bool_(True)))

    hf = h > 0
    gs = jnp.where(hf, sr, -_BIG)
    gi = jnp.where(hf, ir, ibig)

    def first_blk(b, _):
        i0 = pl.multiple_of(b * BLK, BLK)
        cov = j_ref[pl.ds(i0, BLK), :] >= _IOU
        ms = jnp.max(jnp.where(cov, gs, -_BIG), axis=1, keepdims=True)
        ti = jnp.where(cov & (gs == ms), gi, ibig)
        fi = jnp.min(ti, axis=1, keepdims=True)
        sup_ref[pl.ds(i0, BLK), :] = jnp.where(
            fi == ir, 1.0, 0.0).astype(jnp.bfloat16)
        return 0
    lax.fori_loop(0, N // BLK, first_blk, 0)

    mt = sup_ref[...]
    cnt = lax.dot_general(
        mt, jnp.ones((N, 1), jnp.bfloat16), (((0,), (0,)), ((), ())),
        preferred_element_type=jnp.float32)
    ssum = lax.dot_general(
        mt, cd_ref[...], (((0,), (0,)), ((), ())),
        preferred_element_type=jnp.float32)
    inv = jnp.where(cnt > 0.0, 1.0 / jnp.maximum(cnt, 1.0), 0.0)
    avg = (ssum * inv).astype(jnp.bfloat16)

    mcd_ref[...] = jnp.dot(mt, avg, preferred_element_type=jnp.float32)

    num_heads = jnp.sum(jnp.where(cnt > 0.0, 1.0, 0.0))
    max_clique = jnp.max(cnt)
    min_clique = jnp.min(jnp.where(cnt > 0.0, cnt, _BIG))
    lane = lax.broadcasted_iota(jnp.int32, (1, 128), 1)
    stats_ref[...] = (num_heads * (lane == 0).astype(jnp.float32)
                      + max_clique * (lane == 1).astype(jnp.float32)
                      + min_clique * (lane == 2).astype(jnp.float32))


def _merge_pallas(J, sr, sc, CD):
    N, K2 = CD.shape
    vmem_limit = int(min(
        2 * N * N * 4 + N * N * 2 + 8 * N * K2 * 4 + (4 << 20), 60 << 20))
    out_shape = (
        jax.ShapeDtypeStruct((N, K2), jnp.float32),
        jax.ShapeDtypeStruct((1, 128), jnp.float32),
    )
    return pl.pallas_call(
        _merge_kernel,
        out_shape=out_shape,
        grid=(1,),
        in_specs=[
            pl.BlockSpec((N, N), lambda i: (0, 0)),
            pl.BlockSpec((1, N), lambda i: (0, 0)),
            pl.BlockSpec((N, 1), lambda i: (0, 0)),
            pl.BlockSpec((N, K2), lambda i: (0, 0)),
        ],
        out_specs=(
            pl.BlockSpec((N, K2), lambda i: (0, 0)),
            pl.BlockSpec((1, 128), lambda i: (0, 0)),
        ),
        scratch_shapes=[pltpu.VMEM((N, N), jnp.bfloat16)],
        compiler_params=pltpu.CompilerParams(
            dimension_semantics=("arbitrary",),
            vmem_limit_bytes=vmem_limit),
    )(J, sr, sc, CD)


def kernel(S, J, C, D, P):
    N = S.shape[0]
    K = C.shape[1]

    Sf = S.astype(jnp.float32)
    sr = Sf.reshape(1, N)
    sc = Sf.reshape(N, 1)
    CD = jnp.concatenate(
        [C.astype(jnp.bfloat16), D.astype(jnp.bfloat16)], axis=1)

    MCD, stats = _merge_pallas(J.astype(jnp.float32), sr, sc, CD)

    MC = MCD[:, :K].astype(C.dtype)
    MD = MCD[:, K:].astype(D.dtype)

    num_heads = stats[0, 0].astype(jnp.int32)
    max_clique = stats[0, 1].astype(jnp.int32)
    min_clique = stats[0, 2].astype(jnp.int32)
    P_new = (P.at[2].add(1)
              .at[5].add(num_heads)
              .at[6].add(max_clique)
              .at[7].add(min_clique))
    return MC, MD, P_new


# speedup vs baseline: 14.7192x; 1.7934x over previous
import jax
import jax.numpy as jnp
from jax import lax
from jax.experimental import pallas as pl
from jax.experimental.pallas import tpu as pltpu

_IOU = 0.5
_BIG = 1e9


def _merge_kernel(j_ref, sr_ref, cd_ref, mcd_ref, stats_ref, m_ref, cnt_ref):
    N = j_ref.shape[0]
    BLK = min(256, N)
    sr = sr_ref[...]
    ir = lax.broadcasted_iota(jnp.int32, (1, N), 1)
    ibig = jnp.int32(1 << 30)

    def cond(c):
        return jnp.max(c[0]) > 0.0

    def body(c):
        u, f = c
        ub = u > 0.0
        key = jnp.where(ub, sr, -1.0)
        best = jnp.max(key)
        idx = jnp.min(jnp.where(key == best, ir, ibig))
        jrow = j_ref[pl.ds(idx, 1), :]
        newc = ub & (jrow >= _IOU)
        f = jnp.where(newc, idx, f)
        u = jnp.where(newc, 0.0, u)
        return u, f

    _, f = lax.while_loop(
        cond, body,
        (jnp.ones((1, N), jnp.float32), jnp.full((1, N), -1, jnp.int32)))

    def build_blk(b, _):
        i0 = pl.multiple_of(b * BLK, BLK)
        icb = lax.broadcasted_iota(jnp.int32, (BLK, 1), 0) + i0
        mf = jnp.where(icb == f, 1.0, 0.0)
        cnt_ref[pl.ds(i0, BLK), :] = jnp.sum(mf, axis=1, keepdims=True)
        m_ref[pl.ds(i0, BLK), :] = mf.astype(jnp.bfloat16)
        return 0
    lax.fori_loop(0, N // BLK, build_blk, 0)

    cnt = cnt_ref[...]
    ssum = jnp.dot(m_ref[...], cd_ref[...],
                   preferred_element_type=jnp.float32)
    inv = jnp.where(cnt > 0.0, 1.0 / jnp.maximum(cnt, 1.0), 0.0)
    avg = (ssum * inv).astype(jnp.bfloat16)

    mcd_ref[...] = lax.dot_general(
        m_ref[...], avg, (((0,), (0,)), ((), ())),
        preferred_element_type=jnp.float32)

    num_heads = jnp.sum(jnp.where(cnt > 0.0, 1.0, 0.0))
    max_clique = jnp.max(cnt)
    min_clique = jnp.min(jnp.where(cnt > 0.0, cnt, _BIG))
    lane = lax.broadcasted_iota(jnp.int32, (1, 128), 1)
    stats_ref[...] = (num_heads * (lane == 0).astype(jnp.float32)
                      + max_clique * (lane == 1).astype(jnp.float32)
                      + min_clique * (lane == 2).astype(jnp.float32))


def _merge_pallas(J, sr, CD):
    N, K2 = CD.shape
    vmem_limit = int(min(
        2 * N * N * 4 + N * N * 2 + 8 * N * K2 * 4 + (4 << 20), 60 << 20))
    out_shape = (
        jax.ShapeDtypeStruct((N, K2), jnp.float32),
        jax.ShapeDtypeStruct((1, 128), jnp.float32),
    )
    return pl.pallas_call(
        _merge_kernel,
        out_shape=out_shape,
        grid=(1,),
        in_specs=[
            pl.BlockSpec((N, N), lambda i: (0, 0)),
            pl.BlockSpec((1, N), lambda i: (0, 0)),
            pl.BlockSpec((N, K2), lambda i: (0, 0)),
        ],
        out_specs=(
            pl.BlockSpec((N, K2), lambda i: (0, 0)),
            pl.BlockSpec((1, 128), lambda i: (0, 0)),
        ),
        scratch_shapes=[
            pltpu.VMEM((N, N), jnp.bfloat16),
            pltpu.VMEM((N, 1), jnp.float32),
        ],
        compiler_params=pltpu.CompilerParams(
            dimension_semantics=("arbitrary",),
            vmem_limit_bytes=vmem_limit),
    )(J, sr, CD)


def kernel(S, J, C, D, P):
    N = S.shape[0]
    K = C.shape[1]

    sr = S.astype(jnp.float32).reshape(1, N)
    CD = jnp.concatenate(
        [C.astype(jnp.bfloat16), D.astype(jnp.bfloat16)], axis=1)

    MCD, stats = _merge_pallas(J.astype(jnp.float32), sr, CD)

    MC = MCD[:, :K].astype(C.dtype)
    MD = MCD[:, K:].astype(D.dtype)

    num_heads = stats[0, 0].astype(jnp.int32)
    max_clique = stats[0, 1].astype(jnp.int32)
    min_clique = stats[0, 2].astype(jnp.int32)
    P_new = (P.at[2].add(1)
              .at[5].add(num_heads)
              .at[6].add(max_clique)
              .at[7].add(min_clique))
    return MC, MD, P_new
